# Initial kernel scaffold; baseline (speedup 1.0000x reference)
import jax
import jax.numpy as jnp
from jax.experimental import pallas as pl


def kernel(features, vertex_pos_cartesian, out_size, Wp, b):
    raise NotImplementedError("write your pallas kernel here")


# speedup vs baseline: 17.5705x; 17.5705x over previous
import functools

import jax
import jax.numpy as jnp
from jax.experimental import pallas as pl

V = 2048
IN_CHANNELS = 256
K = 4
SCALE = 0.1
D, H, W = 32, 32, 32
N = D * H * W


def _prep_body(features_ref, verts_ref, wp_ref, b_ref, dsph_ref, vu_ref):
    f = features_ref[...]
    wp = wp_ref[...]
    dsph_ref[...] = jax.lax.dot_general(
        f, wp, (((1,), (0,)), ((), ())), preferred_element_type=jnp.float32
    ) + b_ref[...]
    vx = verts_ref[:, 0:1]
    vy = verts_ref[:, 1:2]
    vz = verts_ref[:, 2:3]
    rho = jnp.sqrt(vx * vx + vy * vy + vz * vz)
    inv = 1.0 / jnp.maximum(rho, 1e-12)
    vu_ref[...] = jnp.concatenate([vx * inv, vy * inv, vz * inv], axis=1)


def _main_body(dsph_ref, vu_ref, out_ref, *, block_rows):
    r0 = pl.program_id(0) * block_rows
    row = r0 + jax.lax.broadcasted_iota(jnp.int32, (block_rows, 1), 0)
    gx = (row % W).astype(jnp.float32)
    gy = ((row // W) % H).astype(jnp.float32)
    gz = (row // (H * W)).astype(jnp.float32)
    px = gx - (W - 1) * 0.5
    py = gy - (H - 1) * 0.5
    pz = gz - (D - 1) * 0.5
    rho = jnp.sqrt(px * px + py * py + pz * pz)
    inv = 1.0 / jnp.maximum(rho, 1e-12)
    erx, ery, erz = px * inv, py * inv, pz * inv
    rxy = jnp.sqrt(px * px + py * py)
    st = rxy * inv
    ct = jnp.clip(pz * inv, -1.0, 1.0)
    invxy = 1.0 / jnp.maximum(rxy, 1e-12)
    cp = jnp.where(rxy > 0, px * invxy, 1.0)
    sp = jnp.where(rxy > 0, py * invxy, 0.0)
    etx, ety, etz = ct * cp, ct * sp, -st
    epx, epy = -sp, cp

    er = jnp.concatenate([erx, ery, erz], axis=1)
    vu = vu_ref[...]
    sim = jax.lax.dot_general(
        er, vu, (((1,), (1,)), ((), ())), preferred_element_type=jnp.float32
    )
    sim = jnp.clip(sim, -1.0, 1.0)

    ji = jax.lax.broadcasted_iota(jnp.int32, (block_rows, V), 1)
    wacc = jnp.zeros((block_rows, V), jnp.float32)
    wsum = jnp.zeros((block_rows, 1), jnp.float32)
    simc = sim
    for _ in range(K):
        m = jnp.max(simc, axis=1, keepdims=True)
        cand = jnp.where(simc == m, ji, V)
        j = jnp.min(cand, axis=1, keepdims=True)
        onehot = ji == j
        wk = 1.0 / jnp.maximum((1.0 - m) + 1e-8, 1e-8)
        wacc = wacc + jnp.where(onehot, wk, 0.0)
        simc = jnp.where(onehot, -3.0, simc)
        wsum = wsum + wk

    wn = wacc * (1.0 / wsum)
    dchunk = jax.lax.dot_general(
        wn, dsph_ref[...], (((1,), (0,)), ((), ())),
        preferred_element_type=jnp.float32,
    )
    d_r = dchunk[:, 0:1]
    d_t = dchunk[:, 1:2] * rho
    d_p = dchunk[:, 2:3] * rho * st
    dx = d_r * erx + d_t * etx + d_p * epx
    dy = d_r * ery + d_t * ety + d_p * epy
    dz = d_r * erz + d_t * etz
    out_ref[...] = jnp.concatenate([dx, dy, dz], axis=1) * SCALE


def kernel(features, vertex_pos_cartesian, out_size, Wp, b):
    del out_size
    features = features.reshape(V, IN_CHANNELS)
    verts = vertex_pos_cartesian.reshape(V, 3)
    b2 = b.reshape(1, 3)

    dsph, vu = pl.pallas_call(
        _prep_body,
        out_shape=(
            jax.ShapeDtypeStruct((V, 3), jnp.float32),
            jax.ShapeDtypeStruct((V, 3), jnp.float32),
        ),
    )(features, verts, Wp, b2)

    block_rows = 256
    out = pl.pallas_call(
        functools.partial(_main_body, block_rows=block_rows),
        grid=(N // block_rows,),
        in_specs=[
            pl.BlockSpec((V, 3), lambda i: (0, 0)),
            pl.BlockSpec((V, 3), lambda i: (0, 0)),
        ],
        out_specs=pl.BlockSpec((block_rows, 3), lambda i: (i, 0)),
        out_shape=jax.ShapeDtypeStruct((N, 3), jnp.float32),
    )(dsph, vu)
    return out.reshape(D, H, W, 3)
